# baseline (speedup 1.0000x reference)
import functools

import jax
import jax.numpy as jnp
from jax.experimental import pallas as pl
from jax.experimental.pallas import tpu as pltpu
from jax.experimental.pallas import tpu_sc as plsc

F32 = jnp.float32


def _ceil8(n):
    return (n + 7) // 8 * 8


def _tap_matmul(x, w_ref, starts, L):
    acc = None
    for t, s in enumerate(starts):
        xs = jax.lax.slice(x, (s, 0), (s + L, x.shape[1]))
        p = jnp.dot(xs, w_ref[t], preferred_element_type=F32)
        acc = p if acc is None else acc + p
    return acc


def _s2d_conv_body(x_ref, w_ref, b_ref, o_ref, *, starts, Lc):
    c = pl.program_id(1)
    base = pl.multiple_of(c * Lc, 8)
    x = x_ref[0, pl.ds(base, Lc + _ceil8(starts[-1])), :]
    acc = _tap_matmul(x, w_ref, starts, Lc) + b_ref[:]
    o_ref[0] = jnp.maximum(acc, 0.0)


def _s2d_conv(X, W, b, starts, nc):
    B, P, K = X.shape
    T, _, N = W.shape
    Po = _ceil8((P + nc - 1) // nc) * nc
    Lc = Po // nc
    Xp = Po + _ceil8(starts[-1])
    X = jnp.pad(X, ((0, 0), (0, Xp - P), (0, 0)))
    body = functools.partial(_s2d_conv_body, starts=starts, Lc=Lc)
    return pl.pallas_call(
        body,
        grid=(B, nc),
        in_specs=[
            pl.BlockSpec((1, Xp, K), lambda i, c: (i, 0, 0)),
            pl.BlockSpec((T, K, N), lambda i, c: (0, 0, 0)),
            pl.BlockSpec((1, N), lambda i, c: (0, 0)),
        ],
        out_specs=pl.BlockSpec((1, Lc, N), lambda i, c: (i, c, 0)),
        out_shape=jax.ShapeDtypeStruct((B, Po, N), F32),
    )(X, W, b)


def _enc_dec_core(x, wcr, bcr, wa1r, wb1r, wa2r, wb2r, starts, L, mask, s0):
    y = _tap_matmul(x, wcr, starts, L) + bcr[:]
    y = y * mask
    zpad = jnp.zeros((s0, y.shape[1]), F32)
    for war, wbr in ((wa1r, wb1r), (wa2r, wb2r)):
        xp = jnp.concatenate([zpad, y, zpad], axis=0)
        h = jnp.maximum(xp, 0.0)
        t = _tap_matmul(h, war, starts, L)
        t = jnp.maximum(t, 0.0)
        t = jnp.dot(t, wbr[:], preferred_element_type=F32)
        y = (y + t) * mask
    return jnp.maximum(y, 0.0)


def _mega_body(x_ref, wc, bc, wa1, wb1, wa2, wb2, pv, pvb, cbt, cb,
               wd, bd, da1, db1, da2, db2, wt1, bt1,
               o_ref, loss_ref, perp_ref, cnt_ref, sq_ref,
               *, nb, L, Wp, s0, starts, total):
    bidx = pl.program_id(0)
    col = (jax.lax.broadcasted_iota(jnp.int32, (L, 1), 0) + s0) % Wp
    mask = ((col >= 1) & (col <= Wp - 2)).astype(F32)
    x = x_ref[0]
    f = _enc_dec_core(x, wc, bc, wa1, wb1, wa2, wb2, starts, L, mask, s0)
    z = jnp.dot(f, pv[:], preferred_element_type=F32) + pvb[:]
    cbtv = cbt[:]
    cn = jnp.sum(cbtv * cbtv, axis=0, keepdims=True)
    zn = jnp.sum(z * z, axis=1, keepdims=True)
    mm = jnp.dot(z, cbtv, preferred_element_type=F32)
    dist = zn + cn - 2.0 * mm
    idx = jnp.argmin(dist, axis=1).astype(jnp.int32).reshape(L, 1)
    oh = (jax.lax.broadcasted_iota(jnp.int32, (L, dist.shape[1]), 1)
          == idx).astype(F32)
    q = jnp.dot(oh, cb[:], preferred_element_type=F32)
    e = (q - z) * mask
    sq = jnp.sum(e * e)
    cnt = jnp.sum(oh * mask, axis=0, keepdims=True)

    @pl.when(bidx == 0)
    def _():
        cnt_ref[:] = cnt
        sq_ref[0] = sq

    @pl.when(bidx > 0)
    def _():
        cnt_ref[:] = cnt_ref[:] + cnt
        sq_ref[0] = sq_ref[0] + sq

    @pl.when(bidx == nb - 1)
    def _():
        avg = cnt_ref[:] / total
        ent = jnp.sum(avg * jnp.log(avg + 1e-10))
        perp_ref[:] = jnp.exp(-ent) * jnp.ones((1, 1), F32)
        loss_ref[:] = (0.25 * sq_ref[0] / (total * 64.0)) * jnp.ones((1, 1), F32)

    zq = jnp.zeros((s0, q.shape[1]), F32)
    xq = jnp.concatenate([zq, q * mask, zq], axis=0)
    fd = _enc_dec_core(xq, wd, bd, da1, db1, da2, db2, starts, L, mask, s0)
    zc = jnp.zeros((s0, fd.shape[1]), F32)
    xf = jnp.concatenate([zc, fd, zc], axis=0)
    b1 = bt1[:]
    outs = []
    for pi, (sy, sx) in enumerate(_PH):
        acc = None
        for ti, (uy, ux) in enumerate(_PH):
            s = (sy + uy) * Wp + (sx + ux)
            xs = jax.lax.slice(xf, (s, 0), (s + L, xf.shape[1]))
            p = jnp.dot(xs, wt1[pi, ti], preferred_element_type=F32)
            acc = p if acc is None else acc + p
        outs.append(jnp.maximum(acc + b1, 0.0))
    o_ref[0] = jnp.concatenate(outs, axis=1)


def _mega(X, wc, bc, wa1, wb1, wa2, wb2, pv, pvb, cb,
          wd, bd, da1, db1, da2, db2, wt1, bt1):
    B, P, K = X.shape
    Wp = 58
    s0 = Wp + 1
    L = P - 2 * s0
    starts = tuple(ty * Wp + tx for ty in range(3) for tx in range(3))
    body = functools.partial(_mega_body, nb=B, L=L, Wp=Wp, s0=s0,
                             starts=starts, total=float(B * 56 * 56))
    cst = lambda a: pl.BlockSpec(a.shape, lambda i: (0,) * a.ndim)
    arrs = [X, wc, bc, wa1, wb1, wa2, wb2, pv, pvb, cb.T, cb,
            wd, bd, da1, db1, da2, db2, wt1, bt1]
    in_specs = [pl.BlockSpec((1, P, K), lambda i: (i, 0, 0))]
    in_specs += [cst(a) for a in arrs[1:]]
    scalar_spec = pl.BlockSpec((1, 1), lambda i: (0, 0))
    return pl.pallas_call(
        body,
        grid=(B,),
        in_specs=in_specs,
        out_specs=[
            pl.BlockSpec((1, L, 256), lambda i: (i, 0, 0)),
            scalar_spec, scalar_spec,
        ],
        out_shape=[
            jax.ShapeDtypeStruct((B, L, 256), F32),
            jax.ShapeDtypeStruct((1, 1), F32),
            jax.ShapeDtypeStruct((1, 1), F32),
        ],
        scratch_shapes=[
            pltpu.VMEM((1, 512), F32),
            pltpu.SMEM((1,), F32),
        ],
    )(*arrs)


def _enc_vq_body(x_ref, wc, bc, wa1, wb1, wa2, wb2, pv, pvb, cbt,
                 idx_ref, loss_ref, perp_ref, cnt_ref, sq_ref,
                 *, nb, L, Wp, s0, starts, total):
    bidx = pl.program_id(0)
    col = (jax.lax.broadcasted_iota(jnp.int32, (L, 1), 0) + s0) % Wp
    mask = ((col >= 1) & (col <= Wp - 2)).astype(F32)
    x = x_ref[0]
    f = _enc_dec_core(x, wc, bc, wa1, wb1, wa2, wb2, starts, L, mask, s0)
    z = jnp.dot(f, pv[:], preferred_element_type=F32) + pvb[:]
    cbtv = cbt[:]
    cn = jnp.sum(cbtv * cbtv, axis=0, keepdims=True)
    zn = jnp.sum(z * z, axis=1, keepdims=True)
    mm = jnp.dot(z, cbtv, preferred_element_type=F32)
    dist = zn + cn - 2.0 * mm
    idx = jnp.argmin(dist, axis=1).astype(jnp.int32).reshape(L, 1)
    dmin = jnp.min(dist, axis=1, keepdims=True)
    sq = jnp.sum(dmin * mask)
    oh = (jax.lax.broadcasted_iota(jnp.int32, (L, dist.shape[1]), 1)
          == idx).astype(F32)
    cnt = jnp.sum(oh * mask, axis=0, keepdims=True)
    imask = (mask > 0.0)
    idx_ref[0] = jnp.where(imask, idx, 512).astype(F32)

    @pl.when(bidx == 0)
    def _():
        cnt_ref[:] = cnt
        sq_ref[0] = sq

    @pl.when(bidx > 0)
    def _():
        cnt_ref[:] = cnt_ref[:] + cnt
        sq_ref[0] = sq_ref[0] + sq

    @pl.when(bidx == nb - 1)
    def _():
        avg = cnt_ref[:] / total
        ent = jnp.sum(avg * jnp.log(avg + 1e-10))
        perp_ref[:] = jnp.exp(-ent) * jnp.ones((1, 1), F32)
        loss_ref[:] = (0.25 * sq_ref[0] / (total * 64.0)) * jnp.ones((1, 1), F32)


def _enc_vq(X, wc, bc, wa1, wb1, wa2, wb2, pv, pvb, cb):
    B, P, K = X.shape
    Wp = 58
    s0 = Wp + 1
    L = P - 2 * s0
    starts = tuple(ty * Wp + tx for ty in range(3) for tx in range(3))
    body = functools.partial(_enc_vq_body, nb=B, L=L, Wp=Wp, s0=s0,
                             starts=starts, total=float(B * 56 * 56))
    cst = lambda a: pl.BlockSpec(a.shape, lambda i: (0,) * a.ndim)
    arrs = [X, wc, bc, wa1, wb1, wa2, wb2, pv, pvb, cb.T]
    in_specs = [pl.BlockSpec((1, P, K), lambda i: (i, 0, 0))]
    in_specs += [cst(a) for a in arrs[1:]]
    scalar_spec = pl.BlockSpec((1, 1), lambda i: (0, 0))
    return pl.pallas_call(
        body,
        grid=(B,),
        in_specs=in_specs,
        out_specs=[
            pl.BlockSpec((1, L, 1), lambda i: (i, 0, 0)),
            scalar_spec, scalar_spec,
        ],
        out_shape=[
            jax.ShapeDtypeStruct((B, L, 1), F32),
            jax.ShapeDtypeStruct((1, 1), F32),
            jax.ShapeDtypeStruct((1, 1), F32),
        ],
        scratch_shapes=[
            pltpu.VMEM((1, 512), F32),
            pltpu.SMEM((1,), F32),
        ],
    )(*arrs)


def _sc_gather(table, idx):
    info = plsc.get_sparse_core_info()
    NC, NS = info.num_cores, info.num_subcores
    NW = NC * NS
    Bt = idx.shape[0]
    bpw = Bt // NW
    nch = bpw // 128
    mesh = plsc.VectorSubcoreMesh(core_axis_name="c", subcore_axis_name="s")

    @functools.partial(
        pl.kernel, mesh=mesh,
        out_type=jax.ShapeDtypeStruct((Bt, 128), F32),
        scratch_types=[
            pltpu.VMEM((bpw,), jnp.int32),
            pltpu.VMEM((bpw, 128), F32),
            pltpu.SemaphoreType.DMA,
        ],
    )
    def k(table_hbm, idx_hbm, out_hbm, idx_v, rows_v, sem):
        wid = jax.lax.axis_index("s") * NC + jax.lax.axis_index("c")
        base = wid * bpw
        pltpu.sync_copy(idx_hbm.at[pl.ds(base, bpw)], idx_v)
        for j in range(nch):
            pltpu.async_copy(
                table_hbm.at[idx_v.at[pl.ds(j * 128, 128)]],
                rows_v.at[pl.ds(j * 128, 128)], sem).wait()
        pltpu.sync_copy(rows_v, out_hbm.at[pl.ds(base, bpw)])

    return k(table, idx)


def _dec_body(q_ref, wd, bd, da1, db1, da2, db2, wt1, bt1, o_ref,
              *, L, Wp, s0, starts):
    col = (jax.lax.broadcasted_iota(jnp.int32, (L, 1), 0) + s0) % Wp
    mask = ((col >= 1) & (col <= Wp - 2)).astype(F32)
    q = q_ref[0]
    zq = jnp.zeros((s0, q.shape[1]), F32)
    xq = jnp.concatenate([zq, q * mask, zq], axis=0)
    fd = _enc_dec_core(xq, wd, bd, da1, db1, da2, db2, starts, L, mask, s0)
    zc = jnp.zeros((s0, fd.shape[1]), F32)
    xf = jnp.concatenate([zc, fd, zc], axis=0)
    b1 = bt1[:]
    outs = []
    for pi, (sy, sx) in enumerate(_PH):
        acc = None
        for ti, (uy, ux) in enumerate(_PH):
            s = (sy + uy) * Wp + (sx + ux)
            xs = jax.lax.slice(xf, (s, 0), (s + L, xf.shape[1]))
            p = jnp.dot(xs, wt1[pi, ti], preferred_element_type=F32)
            acc = p if acc is None else acc + p
        outs.append(jnp.maximum(acc + b1, 0.0))
    o_ref[0] = jnp.concatenate(outs, axis=1)


def _dec(Q, wd, bd, da1, db1, da2, db2, wt1, bt1):
    B, L, K = Q.shape
    Wp = 58
    s0 = Wp + 1
    starts = tuple(ty * Wp + tx for ty in range(3) for tx in range(3))
    body = functools.partial(_dec_body, L=L, Wp=Wp, s0=s0, starts=starts)
    cst = lambda a: pl.BlockSpec(a.shape, lambda i: (0,) * a.ndim)
    arrs = [Q, wd, bd, da1, db1, da2, db2, wt1, bt1]
    in_specs = [pl.BlockSpec((1, L, K), lambda i: (i, 0, 0))]
    in_specs += [cst(a) for a in arrs[1:]]
    return pl.pallas_call(
        body,
        grid=(B,),
        in_specs=in_specs,
        out_specs=pl.BlockSpec((1, L, 256), lambda i: (i, 0, 0)),
        out_shape=jax.ShapeDtypeStruct((B, L, 256), F32),
    )(*arrs)


def _stage_body(refs, *, starts, L, Wp, s0, enc):
    if enc:
        (x_ref, wc_ref, bc_ref, wa1_ref, wb1_ref, wa2_ref, wb2_ref,
         pv_ref, pvb_ref, o_ref) = refs
    else:
        (x_ref, wc_ref, bc_ref, wa1_ref, wb1_ref, wa2_ref, wb2_ref,
         o_ref) = refs
    x = x_ref[0]
    y = _tap_matmul(x, wc_ref, starts, L) + bc_ref[:]
    col = (jax.lax.broadcasted_iota(jnp.int32, (L, 1), 0) + s0) % Wp
    mask = ((col >= 1) & (col <= Wp - 2)).astype(F32)
    y = y * mask
    zpad = jnp.zeros((s0, y.shape[1]), F32)
    for wa_ref, wb_ref in ((wa1_ref, wb1_ref), (wa2_ref, wb2_ref)):
        xp = jnp.concatenate([zpad, y, zpad], axis=0)
        h = jnp.maximum(xp, 0.0)
        t = _tap_matmul(h, wa_ref, starts, L)
        t = jnp.maximum(t, 0.0)
        t = jnp.dot(t, wb_ref[:], preferred_element_type=F32)
        y = (y + t) * mask
    f = jnp.maximum(y, 0.0)
    if enc:
        o_ref[0] = jnp.dot(f, pv_ref[:], preferred_element_type=F32) + pvb_ref[:]
    else:
        o_ref[0] = f


def _stage(X, wc, bc, wa1, wb1, wa2, wb2, pv=None, pvb=None):
    B, P, K = X.shape
    Wp = 58
    s0 = Wp + 1
    L = P - 2 * s0
    starts = tuple(ty * Wp + tx for ty in range(3) for tx in range(3))
    enc = pv is not None
    Nout = pv.shape[1] if enc else wc.shape[2]

    def body(*refs):
        _stage_body(refs, starts=starts, L=L, Wp=Wp, s0=s0, enc=enc)

    in_arrays = [X, wc, bc, wa1, wb1, wa2, wb2]
    in_specs = [
        pl.BlockSpec((1, P, K), lambda i: (i, 0, 0)),
        pl.BlockSpec(wc.shape, lambda i: (0, 0, 0)),
        pl.BlockSpec(bc.shape, lambda i: (0, 0)),
        pl.BlockSpec(wa1.shape, lambda i: (0, 0, 0)),
        pl.BlockSpec(wb1.shape, lambda i: (0, 0)),
        pl.BlockSpec(wa2.shape, lambda i: (0, 0, 0)),
        pl.BlockSpec(wb2.shape, lambda i: (0, 0)),
    ]
    if enc:
        in_arrays += [pv, pvb]
        in_specs += [
            pl.BlockSpec(pv.shape, lambda i: (0, 0)),
            pl.BlockSpec(pvb.shape, lambda i: (0, 0)),
        ]
    return pl.pallas_call(
        body,
        grid=(B,),
        in_specs=in_specs,
        out_specs=pl.BlockSpec((1, L, Nout), lambda i: (i, 0, 0)),
        out_shape=jax.ShapeDtypeStruct((B, L, Nout), F32),
    )(*in_arrays)


_PH = ((0, 0), (0, 1), (1, 0), (1, 1))


def _convt_body(x_ref, w_ref, b_ref, o_ref, *, Wp, Lc, relu):
    c = pl.program_id(1)
    base = pl.multiple_of(c * Lc, 8)
    x = x_ref[0, pl.ds(base, Lc + _ceil8(2 * Wp + 2)), :]
    b = b_ref[:]
    outs = []
    for pi, (sy, sx) in enumerate(_PH):
        acc = None
        for ti, (uy, ux) in enumerate(_PH):
            s = (sy + uy) * Wp + (sx + ux)
            xs = jax.lax.slice(x, (s, 0), (s + Lc, x.shape[1]))
            p = jnp.dot(xs, w_ref[pi, ti], preferred_element_type=F32)
            acc = p if acc is None else acc + p
        acc = acc + b
        if relu:
            acc = jnp.maximum(acc, 0.0)
        outs.append(acc)
    o_ref[0] = jnp.concatenate(outs, axis=1)


def _convt(X, W, b, Wp, relu, nc):
    B, P, K = X.shape
    N = W.shape[-1]
    Po = _ceil8((P + nc - 1) // nc) * nc
    Lc = Po // nc
    Xp = Po + _ceil8(2 * Wp + 2)
    X = jnp.pad(X, ((0, 0), (0, Xp - P), (0, 0)))
    body = functools.partial(_convt_body, Wp=Wp, Lc=Lc, relu=relu)
    return pl.pallas_call(
        body,
        grid=(B, nc),
        in_specs=[
            pl.BlockSpec((1, Xp, K), lambda i, c: (i, 0, 0)),
            pl.BlockSpec(W.shape, lambda i, c: (0, 0, 0, 0)),
            pl.BlockSpec(b.shape, lambda i, c: (0, 0)),
        ],
        out_specs=pl.BlockSpec((1, Lc, 4 * N), lambda i, c: (i, c, 0)),
        out_shape=jax.ShapeDtypeStruct((B, Po, 4 * N), F32),
    )(X, W, b)


def _vq_body(z_ref, cbt_ref, cb_ref, q_ref, loss_ref, perp_ref,
             cnt_ref, sq_ref, *, nb, rows, total, emb):
    b = pl.program_id(0)
    z = z_ref[0]
    cbt = cbt_ref[:]
    cn = jnp.sum(cbt * cbt, axis=0, keepdims=True)
    zn = jnp.sum(z * z, axis=1, keepdims=True)
    mm = jnp.dot(z, cbt, preferred_element_type=F32)
    d = zn + cn - 2.0 * mm
    idx = jnp.argmin(d, axis=1).astype(jnp.int32).reshape(rows, 1)
    oh = (jax.lax.broadcasted_iota(jnp.int32, (rows, d.shape[1]), 1)
          == idx).astype(F32)
    q = jnp.dot(oh, cb_ref[:], preferred_element_type=F32)
    q_ref[0] = q
    d = q - z
    sq = jnp.sum(d * d)
    cnt = jnp.sum(oh, axis=0, keepdims=True)

    @pl.when(b == 0)
    def _():
        cnt_ref[:] = cnt
        sq_ref[0] = sq

    @pl.when(b > 0)
    def _():
        cnt_ref[:] = cnt_ref[:] + cnt
        sq_ref[0] = sq_ref[0] + sq

    @pl.when(b == nb - 1)
    def _():
        avg = cnt_ref[:] / total
        ent = jnp.sum(avg * jnp.log(avg + 1e-10))
        perp_ref[:] = jnp.exp(-ent) * jnp.ones((1, 1), F32)
        loss_ref[:] = (0.25 * sq_ref[0] / (total * emb)) * jnp.ones((1, 1), F32)


def _vq(z, cb):
    B, R, E = z.shape
    V = cb.shape[0]
    body = functools.partial(_vq_body, nb=B, rows=R, total=float(B * R),
                             emb=float(E))
    scalar_spec = pl.BlockSpec((1, 1), lambda i: (0, 0))
    return pl.pallas_call(
        body,
        grid=(B,),
        in_specs=[
            pl.BlockSpec((1, R, E), lambda i: (i, 0, 0)),
            pl.BlockSpec((E, V), lambda i: (0, 0)),
            pl.BlockSpec((V, E), lambda i: (0, 0)),
        ],
        out_specs=[
            pl.BlockSpec((1, R, E), lambda i: (i, 0, 0)),
            scalar_spec,
            scalar_spec,
        ],
        out_shape=[
            jax.ShapeDtypeStruct((B, R, E), F32),
            jax.ShapeDtypeStruct((1, 1), F32),
            jax.ShapeDtypeStruct((1, 1), F32),
        ],
        scratch_shapes=[
            pltpu.VMEM((1, V), F32),
            pltpu.SMEM((1,), F32),
        ],
    )(z, cb.T, cb)


def _w9(w):
    return w.transpose(2, 3, 1, 0).reshape(9, w.shape[1], w.shape[0])


def _w_s2d(w):
    O, I = w.shape[0], w.shape[1]
    wt = w.transpose(2, 3, 1, 0).reshape(2, 2, 2, 2, I, O)
    return wt.transpose(0, 2, 1, 3, 4, 5).reshape(4, 4 * I, O)


def _w_convt(w):
    cols = []
    for sy, sx in _PH:
        taps = [w[:, :, 3 - sy - 2 * uy, 3 - sx - 2 * ux] for uy, ux in _PH]
        cols.append(jnp.stack(taps, axis=0))
    return jnp.stack(cols, axis=0)


def _s2d(a):
    B, H, _, C = a.shape
    M = H // 2 + 1
    ap = jnp.pad(a, ((0, 0), (1, 1), (1, 1), (0, 0)))
    ap = ap.reshape(B, M, 2, M, 2, C).transpose(0, 1, 3, 2, 4, 5)
    return ap.reshape(B, M * M, 4 * C)


def _unflat(o, Mp, M):
    B, _, N = o.shape
    return o[:, :Mp * Mp].reshape(B, Mp, Mp, N)[:, :M, :M, :]


def _pad_flat(a):
    B, H, _, C = a.shape
    ap = jnp.pad(a, ((0, 0), (1, 1), (1, 1), (0, 0)))
    return ap.reshape(B, (H + 2) * (H + 2), C)


def _interior(o, Mp, M):
    B, L, N = o.shape
    s0 = Mp + 1
    o = jnp.pad(o, ((0, 0), (s0, Mp * Mp - s0 - L), (0, 0)))
    return o.reshape(B, Mp, Mp, N)[:, 1:M + 1, 1:M + 1, :]


def _d2s(o, Mp, M, N):
    B = o.shape[0]
    a = o[:, :Mp * Mp].reshape(B, Mp, Mp, 2, 2, N)[:, :M, :M]
    return a.transpose(0, 1, 3, 2, 4, 5).reshape(B, 2 * M, 2 * M, N)


@jax.jit
def kernel(x, e1_w, e1_b, e2_w, e2_b, e3_w, e3_b, er1_w1, er1_w2,
           er2_w1, er2_w2, pv_w, pv_b, codebook, d1_w, d1_b, dr1_w1,
           dr1_w2, dr2_w1, dr2_w2, dt1_w, dt1_b, dt2_w, dt2_b):
    B = x.shape[0]
    xt = x.transpose(0, 2, 3, 1)
    X1 = _s2d(xt)
    o1 = _s2d_conv(X1, _w_s2d(e1_w), e1_b.reshape(1, -1),
                   starts=(0, 1, 113, 114), nc=4)
    a1 = _unflat(o1, 113, 112)
    X2 = _s2d(a1)
    o2 = _s2d_conv(X2, _w_s2d(e2_w), e2_b.reshape(1, -1),
                   starts=(0, 1, 57, 58), nc=1)
    a2 = _unflat(o2, 57, 56)
    X3 = _pad_flat(a2)
    idxf, loss, perp = _enc_vq(
        X3, _w9(e3_w), e3_b.reshape(1, -1),
        _w9(er1_w1), er1_w2[:, :, 0, 0].T,
        _w9(er2_w1), er2_w2[:, :, 0, 0].T,
        pv_w[:, :, 0, 0].T, pv_b.reshape(1, -1), codebook)
    L3 = idxf.shape[1]
    nrows = B * L3
    npad = 32 * 896
    idx32 = jnp.pad(idxf.reshape(nrows).astype(jnp.int32),
                    (0, npad - nrows), constant_values=512)
    table = jnp.pad(codebook, ((0, 8), (0, 64)))
    qrows = _sc_gather(table, idx32)
    Q = qrows[:nrows, :64].reshape(B, L3, 64)
    p1 = _dec(Q, _w9(d1_w), d1_b.reshape(1, -1),
              _w9(dr1_w1), dr1_w2[:, :, 0, 0].T,
              _w9(dr2_w1), dr2_w2[:, :, 0, 0].T,
              _w_convt(dt1_w), dt1_b.reshape(1, -1))
    p1 = jnp.pad(p1, ((0, 0), (0, 58 * 58 - p1.shape[1]), (0, 0)))
    a4 = _d2s(p1, 58, 56, 64)
    Xt2 = _pad_flat(a4)
    p2 = _convt(Xt2, _w_convt(dt2_w), dt2_b.reshape(1, -1),
                Wp=114, relu=False, nc=4)
    xr = _d2s(p2, 114, 112, 3).transpose(0, 3, 1, 2)
    return (loss.reshape(()), xr, perp.reshape(()))


# speedup vs baseline: 1.5359x; 1.5359x over previous
import functools

import jax
import jax.numpy as jnp
from jax.experimental import pallas as pl
from jax.experimental.pallas import tpu as pltpu

F32 = jnp.float32


def _ceil8(n):
    return (n + 7) // 8 * 8


def _tap_matmul(x, w_ref, starts, L):
    acc = None
    for t, s in enumerate(starts):
        xs = jax.lax.slice(x, (s, 0), (s + L, x.shape[1]))
        p = jnp.dot(xs, w_ref[t], preferred_element_type=F32)
        acc = p if acc is None else acc + p
    return acc


def _s2d_conv_body(x_ref, w_ref, b_ref, o_ref, *, starts, Lc):
    c = pl.program_id(1)
    base = pl.multiple_of(c * Lc, 8)
    x = x_ref[0, pl.ds(base, Lc + _ceil8(starts[-1])), :]
    acc = _tap_matmul(x, w_ref, starts, Lc) + b_ref[:]
    o_ref[0] = jnp.maximum(acc, 0.0)


def _s2d_conv(X, W, b, starts, nc):
    B, P, K = X.shape
    T, _, N = W.shape
    Po = _ceil8((P + nc - 1) // nc) * nc
    Lc = Po // nc
    Xp = Po + _ceil8(starts[-1])
    X = jnp.pad(X, ((0, 0), (0, Xp - P), (0, 0)))
    body = functools.partial(_s2d_conv_body, starts=starts, Lc=Lc)
    return pl.pallas_call(
        body,
        grid=(B, nc),
        in_specs=[
            pl.BlockSpec((1, Xp, K), lambda i, c: (i, 0, 0)),
            pl.BlockSpec((T, K, N), lambda i, c: (0, 0, 0)),
            pl.BlockSpec((1, N), lambda i, c: (0, 0)),
        ],
        out_specs=pl.BlockSpec((1, Lc, N), lambda i, c: (i, c, 0)),
        out_shape=jax.ShapeDtypeStruct((B, Po, N), F32),
    )(X, W, b)


def _enc_dec_core(x, wcr, bcr, wa1r, wb1r, wa2r, wb2r, starts, L, mask, s0):
    y = _tap_matmul(x, wcr, starts, L) + bcr[:]
    y = y * mask
    zpad = jnp.zeros((s0, y.shape[1]), F32)
    for war, wbr in ((wa1r, wb1r), (wa2r, wb2r)):
        xp = jnp.concatenate([zpad, y, zpad], axis=0)
        h = jnp.maximum(xp, 0.0)
        t = _tap_matmul(h, war, starts, L)
        t = jnp.maximum(t, 0.0)
        t = jnp.dot(t, wbr[:], preferred_element_type=F32)
        y = (y + t) * mask
    return jnp.maximum(y, 0.0)


def _mega_body(x_ref, wc, bc, wa1, wb1, wa2, wb2, pv, pvb, cbt, cb,
               wd, bd, da1, db1, da2, db2, wt1, bt1,
               o_ref, loss_ref, perp_ref, cnt_ref, sq_ref,
               *, nb, L, Wp, s0, starts, total):
    bidx = pl.program_id(0)
    col = (jax.lax.broadcasted_iota(jnp.int32, (L, 1), 0) + s0) % Wp
    mask = ((col >= 1) & (col <= Wp - 2)).astype(F32)
    x = x_ref[0]
    f = _enc_dec_core(x, wc, bc, wa1, wb1, wa2, wb2, starts, L, mask, s0)
    z = jnp.dot(f, pv[:], preferred_element_type=F32) + pvb[:]
    cbtv = cbt[:]
    cn = jnp.sum(cbtv * cbtv, axis=0, keepdims=True)
    zn = jnp.sum(z * z, axis=1, keepdims=True)
    mm = jnp.dot(z, cbtv, preferred_element_type=F32)
    dist = zn + cn - 2.0 * mm
    idx = jnp.argmin(dist, axis=1).astype(jnp.int32).reshape(L, 1)
    oh = (jax.lax.broadcasted_iota(jnp.int32, (L, dist.shape[1]), 1)
          == idx).astype(F32)
    q = jnp.dot(oh, cb[:], preferred_element_type=F32)
    e = (q - z) * mask
    sq = jnp.sum(e * e)
    cnt = jnp.sum(oh * mask, axis=0, keepdims=True)

    @pl.when(bidx == 0)
    def _():
        cnt_ref[:] = cnt
        sq_ref[0] = sq

    @pl.when(bidx > 0)
    def _():
        cnt_ref[:] = cnt_ref[:] + cnt
        sq_ref[0] = sq_ref[0] + sq

    @pl.when(bidx == nb - 1)
    def _():
        avg = cnt_ref[:] / total
        ent = jnp.sum(avg * jnp.log(avg + 1e-10))
        perp_ref[:] = jnp.exp(-ent) * jnp.ones((1, 1), F32)
        loss_ref[:] = (0.25 * sq_ref[0] / (total * 64.0)) * jnp.ones((1, 1), F32)

    zq = jnp.zeros((s0, q.shape[1]), F32)
    xq = jnp.concatenate([zq, q * mask, zq], axis=0)
    fd = _enc_dec_core(xq, wd, bd, da1, db1, da2, db2, starts, L, mask, s0)
    zc = jnp.zeros((s0, fd.shape[1]), F32)
    xf = jnp.concatenate([zc, fd, zc], axis=0)
    b1 = bt1[:]
    outs = []
    for pi, (sy, sx) in enumerate(_PH):
        acc = None
        for ti, (uy, ux) in enumerate(_PH):
            s = (sy + uy) * Wp + (sx + ux)
            xs = jax.lax.slice(xf, (s, 0), (s + L, xf.shape[1]))
            p = jnp.dot(xs, wt1[pi, ti], preferred_element_type=F32)
            acc = p if acc is None else acc + p
        outs.append(jnp.maximum(acc + b1, 0.0))
    o_ref[0] = jnp.concatenate(outs, axis=1)


def _mega(X, wc, bc, wa1, wb1, wa2, wb2, pv, pvb, cb,
          wd, bd, da1, db1, da2, db2, wt1, bt1):
    B, P, K = X.shape
    Wp = 58
    s0 = Wp + 1
    L = P - 2 * s0
    starts = tuple(ty * Wp + tx for ty in range(3) for tx in range(3))
    body = functools.partial(_mega_body, nb=B, L=L, Wp=Wp, s0=s0,
                             starts=starts, total=float(B * 56 * 56))
    cst = lambda a: pl.BlockSpec(a.shape, lambda i: (0,) * a.ndim)
    arrs = [X, wc, bc, wa1, wb1, wa2, wb2, pv, pvb, cb.T, cb,
            wd, bd, da1, db1, da2, db2, wt1, bt1]
    in_specs = [pl.BlockSpec((1, P, K), lambda i: (i, 0, 0))]
    in_specs += [cst(a) for a in arrs[1:]]
    scalar_spec = pl.BlockSpec((1, 1), lambda i: (0, 0))
    return pl.pallas_call(
        body,
        grid=(B,),
        in_specs=in_specs,
        out_specs=[
            pl.BlockSpec((1, L, 256), lambda i: (i, 0, 0)),
            scalar_spec, scalar_spec,
        ],
        out_shape=[
            jax.ShapeDtypeStruct((B, L, 256), F32),
            jax.ShapeDtypeStruct((1, 1), F32),
            jax.ShapeDtypeStruct((1, 1), F32),
        ],
        scratch_shapes=[
            pltpu.VMEM((1, 512), F32),
            pltpu.SMEM((1,), F32),
        ],
    )(*arrs)


_PH = ((0, 0), (0, 1), (1, 0), (1, 1))


def _convt_body(x_ref, w_ref, b_ref, o_ref, *, Wp, Lc, relu):
    c = pl.program_id(1)
    base = pl.multiple_of(c * Lc, 8)
    x = x_ref[0, pl.ds(base, Lc + _ceil8(2 * Wp + 2)), :]
    b = b_ref[:]
    outs = []
    for pi, (sy, sx) in enumerate(_PH):
        acc = None
        for ti, (uy, ux) in enumerate(_PH):
            s = (sy + uy) * Wp + (sx + ux)
            xs = jax.lax.slice(x, (s, 0), (s + Lc, x.shape[1]))
            p = jnp.dot(xs, w_ref[pi, ti], preferred_element_type=F32)
            acc = p if acc is None else acc + p
        acc = acc + b
        if relu:
            acc = jnp.maximum(acc, 0.0)
        outs.append(acc)
    o_ref[0] = jnp.concatenate(outs, axis=1)


def _convt(X, W, b, Wp, relu, nc):
    B, P, K = X.shape
    N = W.shape[-1]
    Po = _ceil8((P + nc - 1) // nc) * nc
    Lc = Po // nc
    Xp = Po + _ceil8(2 * Wp + 2)
    X = jnp.pad(X, ((0, 0), (0, Xp - P), (0, 0)))
    body = functools.partial(_convt_body, Wp=Wp, Lc=Lc, relu=relu)
    return pl.pallas_call(
        body,
        grid=(B, nc),
        in_specs=[
            pl.BlockSpec((1, Xp, K), lambda i, c: (i, 0, 0)),
            pl.BlockSpec(W.shape, lambda i, c: (0, 0, 0, 0)),
            pl.BlockSpec(b.shape, lambda i, c: (0, 0)),
        ],
        out_specs=pl.BlockSpec((1, Lc, 4 * N), lambda i, c: (i, c, 0)),
        out_shape=jax.ShapeDtypeStruct((B, Po, 4 * N), F32),
    )(X, W, b)


def _w9(w):
    return w.transpose(2, 3, 1, 0).reshape(9, w.shape[1], w.shape[0])


def _w_s2d(w):
    O, I = w.shape[0], w.shape[1]
    wt = w.transpose(2, 3, 1, 0).reshape(2, 2, 2, 2, I, O)
    return wt.transpose(0, 2, 1, 3, 4, 5).reshape(4, 4 * I, O)


def _w_convt(w):
    cols = []
    for sy, sx in _PH:
        taps = [w[:, :, 3 - sy - 2 * uy, 3 - sx - 2 * ux] for uy, ux in _PH]
        cols.append(jnp.stack(taps, axis=0))
    return jnp.stack(cols, axis=0)


def _s2d(a):
    B, H, _, C = a.shape
    M = H // 2 + 1
    ap = jnp.pad(a, ((0, 0), (1, 1), (1, 1), (0, 0)))
    ap = ap.reshape(B, M, 2, M, 2, C).transpose(0, 1, 3, 2, 4, 5)
    return ap.reshape(B, M * M, 4 * C)


def _unflat(o, Mp, M):
    B, _, N = o.shape
    return o[:, :Mp * Mp].reshape(B, Mp, Mp, N)[:, :M, :M, :]


def _pad_flat(a):
    B, H, _, C = a.shape
    ap = jnp.pad(a, ((0, 0), (1, 1), (1, 1), (0, 0)))
    return ap.reshape(B, (H + 2) * (H + 2), C)


def _d2s(o, Mp, M, N):
    B = o.shape[0]
    a = o[:, :Mp * Mp].reshape(B, Mp, Mp, 2, 2, N)[:, :M, :M]
    return a.transpose(0, 1, 3, 2, 4, 5).reshape(B, 2 * M, 2 * M, N)


@jax.jit
def kernel(x, e1_w, e1_b, e2_w, e2_b, e3_w, e3_b, er1_w1, er1_w2,
           er2_w1, er2_w2, pv_w, pv_b, codebook, d1_w, d1_b, dr1_w1,
           dr1_w2, dr2_w1, dr2_w2, dt1_w, dt1_b, dt2_w, dt2_b):
    B = x.shape[0]
    xt = x.transpose(0, 2, 3, 1)
    X1 = _s2d(xt)
    o1 = _s2d_conv(X1, _w_s2d(e1_w), e1_b.reshape(1, -1),
                   starts=(0, 1, 113, 114), nc=4)
    a1 = _unflat(o1, 113, 112)
    X2 = _s2d(a1)
    o2 = _s2d_conv(X2, _w_s2d(e2_w), e2_b.reshape(1, -1),
                   starts=(0, 1, 57, 58), nc=1)
    a2 = _unflat(o2, 57, 56)
    X3 = _pad_flat(a2)
    p1, loss, perp = _mega(
        X3, _w9(e3_w), e3_b.reshape(1, -1),
        _w9(er1_w1), er1_w2[:, :, 0, 0].T,
        _w9(er2_w1), er2_w2[:, :, 0, 0].T,
        pv_w[:, :, 0, 0].T, pv_b.reshape(1, -1), codebook,
        _w9(d1_w), d1_b.reshape(1, -1),
        _w9(dr1_w1), dr1_w2[:, :, 0, 0].T,
        _w9(dr2_w1), dr2_w2[:, :, 0, 0].T,
        _w_convt(dt1_w), dt1_b.reshape(1, -1))
    p1 = jnp.pad(p1, ((0, 0), (0, 58 * 58 - p1.shape[1]), (0, 0)))
    a4 = _d2s(p1, 58, 56, 64)
    Xt2 = _pad_flat(a4)
    p2 = _convt(Xt2, _w_convt(dt2_w), dt2_b.reshape(1, -1),
                Wp=114, relu=False, nc=4)
    xr = _d2s(p2, 114, 112, 3).transpose(0, 3, 1, 2)
    return (loss.reshape(()), xr, perp.reshape(()))
